# Initial kernel scaffold; baseline (speedup 1.0000x reference)
import jax
import jax.numpy as jnp
from jax.experimental import pallas as pl


def kernel(x, edge_index, W1_l, b1_l, W1_r, W2_l, b2_l, W2_r, Wc, bc):
    raise NotImplementedError("write your pallas kernel here")


# speedup vs baseline: 16.0632x; 16.0632x over previous
import functools

import jax
import jax.numpy as jnp
from jax import lax
from jax.experimental import pallas as pl
from jax.experimental.pallas import tpu as pltpu
from jax.experimental.pallas import tpu_sc as plsc

N = 10000
E = 320000
D = 128
H1 = 64
H2 = 32
C = 4

NC = 2
NS = 16
NW = NC * NS
CHUNK = 128
NCH = -(-E // (NW * CHUNK))
EPAD = NW * CHUNK * NCH
NPAD = 10240
RPW = NPAD // NS
DW = 16


def _make_sc_agg(d, with_deg):
    scratch = [
        pltpu.VMEM_SHARED((NPAD, d), jnp.float32),
        pltpu.VMEM((NCH, CHUNK), jnp.int32),
        pltpu.VMEM((NCH, CHUNK), jnp.int32),
        pltpu.VMEM((2, CHUNK, d), jnp.float32),
        pltpu.VMEM((CHUNK, d), jnp.float32),
        pltpu.SemaphoreType.DMA,
    ]
    out_type = jax.ShapeDtypeStruct((NC, NPAD, d), jnp.float32)
    if with_deg:
        out_type = (out_type,
                    jax.ShapeDtypeStruct((NC, NPAD, DW), jnp.float32))
        scratch += [
            pltpu.VMEM_SHARED((NPAD, DW), jnp.float32),
            pltpu.VMEM((CHUNK, DW), jnp.float32),
            pltpu.VMEM((CHUNK, DW), jnp.float32),
        ]
    mesh = plsc.VectorSubcoreMesh(core_axis_name="c", subcore_axis_name="s")

    def body(p_hbm, srcg, dstg, *refs):
        if with_deg:
            (out_hbm, deg_hbm, acc_sh, src_v, dst_v, rows_v, zbuf, gsem,
             deg_sh, ones_v, zbuf16) = refs
        else:
            out_hbm, acc_sh, src_v, dst_v, rows_v, zbuf, gsem = refs
        cid = lax.axis_index("c")
        sid = lax.axis_index("s")
        wid = sid * NC + cid

        pltpu.sync_copy(srcg.at[wid], src_v)
        pltpu.sync_copy(dstg.at[wid], dst_v)

        z16 = jnp.zeros((16,), jnp.float32)
        dpv = d // 16

        def zfill(i, carry):
            zbuf[i // dpv, pl.ds((i % dpv) * 16, 16)] = z16
            return carry
        lax.fori_loop(0, CHUNK * dpv, zfill, 0)
        base = sid * RPW
        for k in range(RPW // CHUNK):
            pltpu.sync_copy(zbuf, acc_sh.at[pl.ds(base + k * CHUNK, CHUNK)])
        if with_deg:
            one16 = jnp.ones((16,), jnp.float32)

            def dfill(i, carry):
                zbuf16[i, pl.ds(0, 16)] = z16
                ones_v[i, pl.ds(0, 16)] = one16
                return carry
            lax.fori_loop(0, CHUNK, dfill, 0)
            for k in range(RPW // CHUNK):
                pltpu.sync_copy(zbuf16,
                                deg_sh.at[pl.ds(base + k * CHUNK, CHUNK)])
        plsc.subcore_barrier()

        pltpu.async_copy(p_hbm.at[src_v.at[0]], rows_v.at[0], gsem)

        def step(j, carry):
            slot = lax.rem(j, 2)
            nslot = lax.rem(j + 1, 2)

            @pl.when(j + 1 < NCH)
            def _():
                pltpu.async_copy(p_hbm.at[src_v.at[j + 1]],
                                 rows_v.at[nslot], gsem)
            pltpu.make_async_copy(p_hbm.at[src_v.at[j]],
                                  rows_v.at[slot], gsem).wait()
            pltpu.sync_copy(rows_v.at[slot], acc_sh.at[dst_v.at[j]], add=True)
            if with_deg:
                pltpu.sync_copy(ones_v, deg_sh.at[dst_v.at[j]], add=True)
            return carry
        lax.fori_loop(0, NCH, step, 0)
        plsc.subcore_barrier()

        for k in range(RPW // CHUNK):
            sl = pl.ds(base + k * CHUNK, CHUNK)
            pltpu.sync_copy(acc_sh.at[sl], zbuf)
            pltpu.sync_copy(zbuf, out_hbm.at[cid, sl])
            if with_deg:
                pltpu.sync_copy(deg_sh.at[sl], zbuf16)
                pltpu.sync_copy(zbuf16, deg_hbm.at[cid, sl])

    return pl.kernel(
        body, out_type=out_type, mesh=mesh, scratch_types=scratch,
        compiler_params=pltpu.CompilerParams(use_tc_tiling_on_sc=False))


_sc_agg1 = _make_sc_agg(H1, True)
_sc_agg2 = _make_sc_agg(H2, False)


def _proj1_body(x_ref, wl_ref, wr_ref, p_ref, r_ref):
    xb = x_ref[...]
    p_ref[...] = jnp.dot(xb, wl_ref[...], preferred_element_type=jnp.float32)
    r_ref[...] = jnp.dot(xb, wr_ref[...], preferred_element_type=jnp.float32)


def _proj1(x, wl, wr):
    return pl.pallas_call(
        _proj1_body,
        out_shape=(jax.ShapeDtypeStruct((N, H1), jnp.float32),
                   jax.ShapeDtypeStruct((N, H1), jnp.float32)),
    )(x, wl, wr)


def _comb1_body(s_ref, deg_ref, r1_ref, b1_ref, wl_ref, wr_ref,
                p2_ref, r2_ref):
    deg = jnp.maximum(deg_ref[0] + deg_ref[1], 1.0)
    agg = (s_ref[0] + s_ref[1]) * (1.0 / deg)[:, 0:1]
    h1 = jnp.maximum(agg[:N] + b1_ref[...] + r1_ref[...], 0.0)
    p2_ref[...] = jnp.dot(h1, wl_ref[...], preferred_element_type=jnp.float32)
    r2_ref[...] = jnp.dot(h1, wr_ref[...], preferred_element_type=jnp.float32)


def _comb1(s, deg, r1, b1, wl, wr):
    return pl.pallas_call(
        _comb1_body,
        out_shape=(jax.ShapeDtypeStruct((N, H2), jnp.float32),
                   jax.ShapeDtypeStruct((N, H2), jnp.float32)),
    )(s, deg, r1, b1, wl, wr)


def _comb2_body(t_ref, deg_ref, r2_ref, b2_ref, wc_ref, bc_ref,
                h2_ref, z_ref):
    deg = jnp.maximum(deg_ref[0] + deg_ref[1], 1.0)
    agg = (t_ref[0] + t_ref[1]) * (1.0 / deg)[:, 0:1]
    h2 = jnp.maximum(agg[:N] + b2_ref[...] + r2_ref[...], 0.0)
    h2_ref[...] = h2
    z_ref[...] = (jnp.dot(h2, wc_ref[...], preferred_element_type=jnp.float32)
                  + bc_ref[...])


def _comb2(t, deg, r2, b2, wc, bc):
    return pl.pallas_call(
        _comb2_body,
        out_shape=(jax.ShapeDtypeStruct((N, H2), jnp.float32),
                   jax.ShapeDtypeStruct((N, C), jnp.float32)),
    )(t, deg, r2, b2, wc, bc)


def kernel(x, edge_index, W1_l, b1_l, W1_r, W2_l, b2_l, W2_r, Wc, bc):
    ei = edge_index.astype(jnp.int32)
    npe = EPAD - E
    pad_i = jnp.arange(npe, dtype=jnp.int32)
    pad_src = (pad_i * 97) % N
    pad_dst = N + pad_i % (NPAD - N)
    srcg = jnp.concatenate([ei[0], pad_src]).reshape(NW, NCH, CHUNK)
    dstg = jnp.concatenate([ei[1], pad_dst]).reshape(NW, NCH, CHUNK)

    p1, r1 = _proj1(x, W1_l, W1_r)
    s1, deg = _sc_agg1(p1, srcg, dstg)
    p2, r2 = _comb1(s1, deg, r1, b1_l.reshape(1, H1), W2_l, W2_r)
    t2 = _sc_agg2(p2, srcg, dstg)
    h2, z = _comb2(t2, deg, r2, b2_l.reshape(1, H2), Wc,
                   bc.reshape(1, C))
    return (h2, z)


# speedup vs baseline: 19.4436x; 1.2104x over previous
import functools

import jax
import jax.numpy as jnp
from jax import lax
from jax.experimental import pallas as pl
from jax.experimental.pallas import tpu as pltpu
from jax.experimental.pallas import tpu_sc as plsc

N = 10000
E = 320000
D = 128
H1 = 64
H2 = 32
C = 4

NC = 2
NS = 16
NW = NC * NS
CHUNK = 128
NCH = -(-E // (NW * CHUNK))
EPAD = NW * CHUNK * NCH
NPAD = 10240
RPW = NPAD // NS
DW = 16


NB = 4


def _make_sc_agg(d, with_deg):
    scratch = [
        pltpu.VMEM_SHARED((NPAD, d), jnp.float32),
        pltpu.VMEM((NCH, CHUNK), jnp.int32),
        pltpu.VMEM((NCH, CHUNK), jnp.int32),
        pltpu.VMEM((NB, CHUNK, d), jnp.float32),
        pltpu.VMEM((CHUNK, d), jnp.float32),
        pltpu.SemaphoreType.DMA,
        pltpu.SemaphoreType.DMA,
        pltpu.SemaphoreType.DMA,
    ]
    out_type = jax.ShapeDtypeStruct((NC, NPAD, d), jnp.float32)
    if with_deg:
        out_type = (out_type,
                    jax.ShapeDtypeStruct((NC, NPAD, DW), jnp.float32))
        scratch += [
            pltpu.VMEM_SHARED((NPAD, DW), jnp.float32),
            pltpu.VMEM((CHUNK, DW), jnp.float32),
            pltpu.VMEM((CHUNK, DW), jnp.float32),
            pltpu.SemaphoreType.DMA,
            pltpu.SemaphoreType.DMA,
        ]
    mesh = plsc.VectorSubcoreMesh(core_axis_name="c", subcore_axis_name="s")

    def body(p_hbm, srcg, dstg, *refs):
        if with_deg:
            (out_hbm, deg_hbm, acc_sh, src_v, dst_v, rows_v, zbuf,
             gsem, ssem, dsem, deg_sh, ones_v, zbuf16, osem, esem) = refs
        else:
            (out_hbm, acc_sh, src_v, dst_v, rows_v, zbuf,
             gsem, ssem, dsem) = refs
        cid = lax.axis_index("c")
        sid = lax.axis_index("s")
        wid = sid * NC + cid

        pltpu.sync_copy(srcg.at[wid], src_v)
        pltpu.sync_copy(dstg.at[wid], dst_v)

        z16 = jnp.zeros((16,), jnp.float32)
        dpv = d // 16

        def zfill(i, carry):
            zbuf[i // dpv, pl.ds((i % dpv) * 16, 16)] = z16
            return carry
        lax.fori_loop(0, CHUNK * dpv, zfill, 0)
        base = sid * RPW
        for k in range(RPW // CHUNK):
            pltpu.sync_copy(zbuf, acc_sh.at[pl.ds(base + k * CHUNK, CHUNK)])
        if with_deg:
            one16 = jnp.ones((16,), jnp.float32)

            def dfill(i, carry):
                zbuf16[i, pl.ds(0, 16)] = z16
                ones_v[i, pl.ds(0, 16)] = one16
                return carry
            lax.fori_loop(0, CHUNK, dfill, 0)
            for k in range(RPW // CHUNK):
                pltpu.sync_copy(zbuf16,
                                deg_sh.at[pl.ds(base + k * CHUNK, CHUNK)])
        plsc.subcore_barrier()

        for b in range(NB):
            pltpu.async_copy(p_hbm.at[src_v.at[b]], rows_v.at[b], gsem)

        def step(j, carry):
            slot = lax.rem(j, NB)
            pltpu.make_async_copy(p_hbm.at[src_v.at[j]],
                                  rows_v.at[slot], gsem).wait()
            pltpu.async_copy(rows_v.at[slot], acc_sh.at[dst_v.at[j]],
                             ssem, add=True)
            if with_deg:
                pltpu.async_copy(ones_v, deg_sh.at[dst_v.at[j]],
                                 osem, add=True)

                @pl.when(j >= 1)
                def _():
                    pltpu.make_async_copy(
                        ones_v, deg_sh.at[dst_v.at[0]], osem).wait()

            @pl.when(j + NB < NCH)
            def _():
                pltpu.make_async_copy(
                    rows_v.at[slot], acc_sh.at[dst_v.at[0]], ssem).wait()
                pltpu.async_copy(p_hbm.at[src_v.at[j + NB]],
                                 rows_v.at[slot], gsem)
            return carry
        lax.fori_loop(0, NCH, step, 0)
        for b in range(NB):
            pltpu.make_async_copy(rows_v.at[0], acc_sh.at[dst_v.at[0]],
                                  ssem).wait()
        if with_deg:
            pltpu.make_async_copy(ones_v, deg_sh.at[dst_v.at[0]],
                                  osem).wait()
        plsc.subcore_barrier()

        nd = RPW // CHUNK
        for k in range(nd):
            sl = pl.ds(base + k * CHUNK, CHUNK)
            slot = k % NB
            if k >= NB:
                pltpu.make_async_copy(rows_v.at[0], out_hbm.at[0, sl],
                                      dsem).wait()
            pltpu.sync_copy(acc_sh.at[sl], rows_v.at[slot])
            pltpu.async_copy(rows_v.at[slot], out_hbm.at[cid, sl], dsem)
            if with_deg:
                dbuf = ones_v if k % 2 else zbuf16
                if k >= 2:
                    pltpu.make_async_copy(dbuf, deg_hbm.at[0, sl],
                                          esem).wait()
                pltpu.sync_copy(deg_sh.at[sl], dbuf)
                pltpu.async_copy(dbuf, deg_hbm.at[cid, sl], esem)
        for k in range(min(nd, NB)):
            pltpu.make_async_copy(rows_v.at[0],
                                  out_hbm.at[0, pl.ds(0, CHUNK)], dsem).wait()
        if with_deg:
            for k in range(min(nd, 2)):
                pltpu.make_async_copy(zbuf16,
                                      deg_hbm.at[0, pl.ds(0, CHUNK)],
                                      esem).wait()

    return pl.kernel(
        body, out_type=out_type, mesh=mesh, scratch_types=scratch,
        compiler_params=pltpu.CompilerParams(use_tc_tiling_on_sc=False))


_sc_agg1 = _make_sc_agg(H1, True)
_sc_agg2 = _make_sc_agg(H2, False)


def _proj1_body(x_ref, wl_ref, wr_ref, p_ref, r_ref):
    xb = x_ref[...]
    p_ref[...] = jnp.dot(xb, wl_ref[...], preferred_element_type=jnp.float32)
    r_ref[...] = jnp.dot(xb, wr_ref[...], preferred_element_type=jnp.float32)


def _proj1(x, wl, wr):
    return pl.pallas_call(
        _proj1_body,
        out_shape=(jax.ShapeDtypeStruct((N, H1), jnp.float32),
                   jax.ShapeDtypeStruct((N, H1), jnp.float32)),
    )(x, wl, wr)


def _comb1_body(s_ref, deg_ref, r1_ref, b1_ref, wl_ref, wr_ref,
                p2_ref, r2_ref):
    deg = jnp.maximum(deg_ref[0] + deg_ref[1], 1.0)
    agg = (s_ref[0] + s_ref[1]) * (1.0 / deg)[:, 0:1]
    h1 = jnp.maximum(agg[:N] + b1_ref[...] + r1_ref[...], 0.0)
    p2_ref[...] = jnp.dot(h1, wl_ref[...], preferred_element_type=jnp.float32)
    r2_ref[...] = jnp.dot(h1, wr_ref[...], preferred_element_type=jnp.float32)


def _comb1(s, deg, r1, b1, wl, wr):
    return pl.pallas_call(
        _comb1_body,
        out_shape=(jax.ShapeDtypeStruct((N, H2), jnp.float32),
                   jax.ShapeDtypeStruct((N, H2), jnp.float32)),
    )(s, deg, r1, b1, wl, wr)


def _comb2_body(t_ref, deg_ref, r2_ref, b2_ref, wc_ref, bc_ref,
                h2_ref, z_ref):
    deg = jnp.maximum(deg_ref[0] + deg_ref[1], 1.0)
    agg = (t_ref[0] + t_ref[1]) * (1.0 / deg)[:, 0:1]
    h2 = jnp.maximum(agg[:N] + b2_ref[...] + r2_ref[...], 0.0)
    h2_ref[...] = h2
    z_ref[...] = (jnp.dot(h2, wc_ref[...], preferred_element_type=jnp.float32)
                  + bc_ref[...])


def _comb2(t, deg, r2, b2, wc, bc):
    return pl.pallas_call(
        _comb2_body,
        out_shape=(jax.ShapeDtypeStruct((N, H2), jnp.float32),
                   jax.ShapeDtypeStruct((N, C), jnp.float32)),
    )(t, deg, r2, b2, wc, bc)


def kernel(x, edge_index, W1_l, b1_l, W1_r, W2_l, b2_l, W2_r, Wc, bc):
    ei = edge_index.astype(jnp.int32)
    npe = EPAD - E
    pad_i = jnp.arange(npe, dtype=jnp.int32)
    pad_src = (pad_i * 97) % N
    pad_dst = N + pad_i % (NPAD - N)
    srcg = jnp.concatenate([ei[0], pad_src]).reshape(NW, NCH, CHUNK)
    dstg = jnp.concatenate([ei[1], pad_dst]).reshape(NW, NCH, CHUNK)

    p1, r1 = _proj1(x, W1_l, W1_r)
    s1, deg = _sc_agg1(p1, srcg, dstg)
    p2, r2 = _comb1(s1, deg, r1, b1_l.reshape(1, H1), W2_l, W2_r)
    t2 = _sc_agg2(p2, srcg, dstg)
    h2, z = _comb2(t2, deg, r2, b2_l.reshape(1, H2), Wc,
                   bc.reshape(1, C))
    return (h2, z)
